# baseline (speedup 1.0000x reference)
import jax
import jax.numpy as jnp
from jax.experimental import pallas as pl
from jax.experimental.pallas import tpu as pltpu

_H0, _W0 = 56, 120
_N_CAM, _M_HEADS, _HG, _WG = 6, 4, 25, 25
_P = _HG * _WG
_K = _N_CAM * _H0 * _W0
_KB = 1920
_NK = _K // _KB


def _argmax_body(a0_ref, a1_ref, a2_ref, a3_ref, ood_ref, out_ref,
                 rmax_ref, rval_ref):
    k = pl.program_id(1)

    @pl.when(k == 0)
    def _():
        rmax_ref[...] = jnp.full_like(rmax_ref, -jnp.inf)
        rval_ref[...] = jnp.zeros_like(rval_ref)

    s = a0_ref[0] + a1_ref[0] + a2_ref[0] + a3_ref[0]

    bmax = jnp.max(s, axis=-1, keepdims=True)
    ki = jax.lax.broadcasted_iota(jnp.int32, (_P, _KB), 1)
    cand = jnp.where(s == bmax, ki, _KB)
    bidx = jnp.min(cand, axis=-1, keepdims=True)
    ood_blk = ood_ref[0]
    bval = jnp.sum(jnp.where(ki == bidx, ood_blk, 0.0),
                   axis=-1, keepdims=True)

    upd = bmax > rmax_ref[...]
    rmax_ref[...] = jnp.where(upd, bmax, rmax_ref[...])
    rval_ref[...] = jnp.where(upd, bval, rval_ref[...])

    @pl.when(k == _NK - 1)
    def _():
        out_ref[0] = rval_ref[...]


def _bce_body(mask_ref, y_ref, out_ref):
    m = mask_ref[0]
    t = y_ref[0, 0]
    logp = jnp.maximum(jnp.log(m), -100.0)
    log1mp = jnp.maximum(jnp.log1p(-m), -100.0)

    cell = jax.lax.broadcasted_iota(jnp.int32, (_WG, 8 * _WG), 1) // 8
    row = jax.lax.broadcasted_iota(jnp.int32, (_WG, 8 * _WG), 0)
    g = (cell == row).astype(jnp.float32)
    logp_w = jnp.dot(logp, g, preferred_element_type=jnp.float32)
    log1mp_w = jnp.dot(log1mp, g, preferred_element_type=jnp.float32)
    tc = jnp.dot(g, t, preferred_element_type=jnp.float32)
    out_ref[...] = -(jnp.sum(tc * logp_w)
                     + jnp.sum((8.0 - tc) * log1mp_w))[None, None, None]


def kernel(alpha, y, ood, ood_cam, att0, att1):
    B = y.shape[0]
    ood_flat = ood_cam.reshape(B, 1, _K)

    rval = pl.pallas_call(
        _argmax_body,
        grid=(B, _NK),
        in_specs=[
            pl.BlockSpec((1, _P, _KB),
                         (lambda b, k, m=m: (_M_HEADS * b + m, 0, k)))
            for m in range(_M_HEADS)
        ] + [
            pl.BlockSpec((1, 1, _KB), lambda b, k: (b, 0, k)),
        ],
        out_specs=pl.BlockSpec((1, _P, 1), lambda b, k: (b, 0, 0)),
        out_shape=jax.ShapeDtypeStruct((B, _P, 1), jnp.float32),
        scratch_shapes=[
            pltpu.VMEM((_P, 1), jnp.float32),
            pltpu.VMEM((_P, 1), jnp.float32),
        ],
        compiler_params=pltpu.CompilerParams(
            dimension_semantics=("parallel", "arbitrary"),
            vmem_limit_bytes=56 * 1024 * 1024,
        ),
    )(att0, att0, att0, att0, ood_flat)

    mask = rval.reshape(B, _HG, _WG)

    out = pl.pallas_call(
        _bce_body,
        grid=(B,),
        in_specs=[
            pl.BlockSpec((1, _HG, _WG), lambda b: (b, 0, 0)),
            pl.BlockSpec((1, 1, 8 * _HG, 8 * _WG), lambda b: (b, 0, 0, 0)),
        ],
        out_specs=pl.BlockSpec((1, 1, 1), lambda b: (b, 0, 0)),
        out_shape=jax.ShapeDtypeStruct((B, 1, 1), jnp.float32),
        compiler_params=pltpu.CompilerParams(
            dimension_semantics=("parallel",),
        ),
    )(mask, y)

    return out.sum() / (B * 8 * _HG * 8 * _WG)


# speedup vs baseline: 1.2247x; 1.2247x over previous
import jax
import jax.numpy as jnp
from jax.experimental import pallas as pl
from jax.experimental.pallas import tpu as pltpu

_H0, _W0 = 56, 120
_N_CAM, _M_HEADS, _HG, _WG = 6, 4, 25, 25
_P = _HG * _WG
_K = _N_CAM * _H0 * _W0
_KB = 640
_NKB = _K // _KB
_STEPS = (_NKB + 1) // 2


def _argmax_body(att_ref, ood_ref, max_ref, val_ref,
                 rmax0, rval0, rmax1, rval1):
    k = pl.program_id(1)

    @pl.when(k == 0)
    def _():
        for r in (rmax0, rmax1):
            r[...] = jnp.full_like(r, -jnp.inf)
        for r in (rval0, rval1):
            r[...] = jnp.zeros_like(r)

    x = att_ref[...].reshape(_P, 8, _KB)
    ki = jax.lax.broadcasted_iota(jnp.int32, (_P, _KB), 1)

    for b, (rmax, rval) in enumerate(((rmax0, rval0), (rmax1, rval1))):
        s = (x[:, 4 * b] + x[:, 4 * b + 1]
             + x[:, 4 * b + 2] + x[:, 4 * b + 3])
        bmax = jnp.max(s, axis=-1, keepdims=True)
        cand = jnp.where(s == bmax, ki, _KB)
        bidx = jnp.min(cand, axis=-1, keepdims=True)
        ood_blk = ood_ref[b]
        bval = jnp.sum(jnp.where(ki == bidx, ood_blk, 0.0),
                       axis=-1, keepdims=True)
        upd = bmax > rmax[...]
        rmax[...] = jnp.where(upd, bmax, rmax[...])
        rval[...] = jnp.where(upd, bval, rval[...])

    @pl.when(k == _STEPS - 1)
    def _():
        max_ref[0, 0] = rmax0[...]
        max_ref[0, 1] = rmax1[...]
        val_ref[0, 0] = rval0[...]
        val_ref[0, 1] = rval1[...]


def _bce_body(mask_ref, y_ref, out_ref):
    m = mask_ref[0]
    t = y_ref[0, 0]
    logp = jnp.maximum(jnp.log(m), -100.0)
    log1mp = jnp.maximum(jnp.log1p(-m), -100.0)

    cell = jax.lax.broadcasted_iota(jnp.int32, (_WG, 8 * _WG), 1) // 8
    row = jax.lax.broadcasted_iota(jnp.int32, (_WG, 8 * _WG), 0)
    g = (cell == row).astype(jnp.float32)
    logp_w = jnp.dot(logp, g, preferred_element_type=jnp.float32)
    log1mp_w = jnp.dot(log1mp, g, preferred_element_type=jnp.float32)
    tc = jnp.dot(g, t, preferred_element_type=jnp.float32)
    out_ref[...] = -(jnp.sum(tc * logp_w)
                     + jnp.sum((8.0 - tc) * log1mp_w))[None, None, None]


def kernel(alpha, y, ood, ood_cam, att0, att1):
    B = y.shape[0]
    att2d = att0.transpose(1, 0, 2).reshape(_P * 2 * _M_HEADS, _K)
    ood_flat = ood_cam.reshape(B, 1, _K)

    vmax, vval = pl.pallas_call(
        _argmax_body,
        grid=(2, _STEPS),
        in_specs=[
            pl.BlockSpec((_P * 2 * _M_HEADS, _KB),
                         lambda c, k: (0, jnp.minimum(c * _STEPS + k,
                                                      _NKB - 1))),
            pl.BlockSpec((B, 1, _KB),
                         lambda c, k: (0, 0, jnp.minimum(c * _STEPS + k,
                                                         _NKB - 1))),
        ],
        out_specs=[
            pl.BlockSpec((1, B, _P, 1), lambda c, k: (c, 0, 0, 0)),
            pl.BlockSpec((1, B, _P, 1), lambda c, k: (c, 0, 0, 0)),
        ],
        out_shape=[
            jax.ShapeDtypeStruct((2, B, _P, 1), jnp.float32),
            jax.ShapeDtypeStruct((2, B, _P, 1), jnp.float32),
        ],
        scratch_shapes=[pltpu.VMEM((_P, 1), jnp.float32)
                        for _ in range(4)],
        compiler_params=pltpu.CompilerParams(
            dimension_semantics=("parallel", "arbitrary"),
            vmem_limit_bytes=56 * 1024 * 1024,
        ),
    )(att2d, ood_flat)

    mask = jnp.where(vmax[1] > vmax[0], vval[1], vval[0]).reshape(B, _HG, _WG)

    out = pl.pallas_call(
        _bce_body,
        grid=(B,),
        in_specs=[
            pl.BlockSpec((1, _HG, _WG), lambda b: (b, 0, 0)),
            pl.BlockSpec((1, 1, 8 * _HG, 8 * _WG), lambda b: (b, 0, 0, 0)),
        ],
        out_specs=pl.BlockSpec((1, 1, 1), lambda b: (b, 0, 0)),
        out_shape=jax.ShapeDtypeStruct((B, 1, 1), jnp.float32),
        compiler_params=pltpu.CompilerParams(
            dimension_semantics=("parallel",),
        ),
    )(mask, y)

    return out.sum() / (B * 8 * _HG * 8 * _WG)
